# baseline (speedup 1.0000x reference)
import functools

import jax
import jax.numpy as jnp
from jax import lax
from jax.experimental import pallas as pl
from jax.experimental.pallas import tpu as pltpu
from jax.experimental.pallas import tpu_sc as plsc

_INFO = plsc.get_sparse_core_info()
_NC = _INFO.num_cores
_NS = _INFO.num_subcores
_NW = _NC * _NS


def _embed_lookup(input_ids, table):
    b, h = input_ids.shape
    d = table.shape[1]
    rows_per_w = b // _NW
    mesh = plsc.VectorSubcoreMesh(core_axis_name="c", subcore_axis_name="s")

    @functools.partial(
        pl.kernel,
        mesh=mesh,
        compiler_params=pltpu.CompilerParams(use_tc_tiling_on_sc=False),
        out_type=jax.ShapeDtypeStruct((b, h, d), jnp.float32),
        scratch_types=[
            pltpu.VMEM((rows_per_w, h), jnp.int32),
            pltpu.VMEM((h, d), jnp.float32),
            pltpu.VMEM((h, d), jnp.float32),
            pltpu.SemaphoreType.DMA,
            pltpu.SemaphoreType.DMA,
            pltpu.SemaphoreType.DMA,
            pltpu.SemaphoreType.DMA,
        ],
    )
    def k(ids_hbm, table_hbm, out_hbm, idx_v, buf0, buf1, g0, g1, s0, s1):
        wid = lax.axis_index("s") * _NC + lax.axis_index("c")
        row0 = wid * rows_per_w
        pltpu.sync_copy(ids_hbm.at[pl.ds(row0, rows_per_w)], idx_v)

        def gather(i, buf, sem):
            pltpu.async_copy(table_hbm.at[idx_v.at[i]], buf, sem)

        def gather_wait(i, buf, sem):
            pltpu.make_async_copy(table_hbm.at[idx_v.at[i]], buf, sem).wait()

        def store(i, buf, sem):
            pltpu.async_copy(buf, out_hbm.at[row0 + i], sem)

        def store_wait(i, buf, sem):
            pltpu.make_async_copy(buf, out_hbm.at[row0 + i], sem).wait()

        gather(0, buf0, g0)
        gather(1, buf1, g1)

        def pair(p, _):
            i0 = 2 * p
            gather_wait(i0, buf0, g0)
            store(i0, buf0, s0)
            gather_wait(i0 + 1, buf1, g1)
            store(i0 + 1, buf1, s1)
            store_wait(i0, buf0, s0)
            gather(i0 + 2, buf0, g0)
            store_wait(i0 + 1, buf1, s1)
            gather(i0 + 3, buf1, g1)
            return 0

        lax.fori_loop(0, rows_per_w // 2 - 1, pair, 0)

        i0 = rows_per_w - 2
        gather_wait(i0, buf0, g0)
        store(i0, buf0, s0)
        gather_wait(i0 + 1, buf1, g1)
        store(i0 + 1, buf1, s1)
        store_wait(i0, buf0, s0)
        store_wait(i0 + 1, buf1, s1)

    return k(input_ids, table)


def kernel(input_ids, table):
    return _embed_lookup(input_ids.astype(jnp.int32), table)
